# Initial kernel scaffold; baseline (speedup 1.0000x reference)
import jax
import jax.numpy as jnp
from jax.experimental import pallas as pl


def kernel(x, weight):
    raise NotImplementedError("write your pallas kernel here")


# speedup vs baseline: 5.0484x; 5.0484x over previous
import functools

import jax
import jax.numpy as jnp
from jax import lax
from jax.experimental import pallas as pl
from jax.experimental.pallas import tpu as pltpu
from jax.experimental.pallas import tpu_sc as plsc

PHONEME_SIZE = 1000
D = 64
BATCH = 16384
SEQ = 50

_INFO = plsc.get_sparse_core_info()
_NC = _INFO.num_cores
_NS = _INFO.num_subcores
_NW = _NC * _NS
_CHUNK = 128

_B_TOTAL = BATCH * SEQ
_B_PER_W = _B_TOTAL // _NW
_NCHUNK = _B_PER_W // _CHUNK


@functools.partial(
    pl.kernel,
    out_type=jax.ShapeDtypeStruct((_B_TOTAL, D), jnp.float32),
    mesh=plsc.VectorSubcoreMesh(core_axis_name="c", subcore_axis_name="s"),
    compiler_params=pltpu.CompilerParams(use_tc_tiling_on_sc=False),
    scratch_types=[
        pltpu.VMEM((_NCHUNK, _CHUNK), jnp.int32),
        pltpu.VMEM((_CHUNK, D), jnp.float32),
        pltpu.SemaphoreType.DMA,
    ],
)
def _embed_sc(x_hbm, table_hbm, out_hbm, idx_v, rows_v, sem):
    wid = lax.axis_index("s") * _NC + lax.axis_index("c")
    base = wid * _B_PER_W
    pltpu.sync_copy(x_hbm.at[wid], idx_v)

    def body(j, carry):
        pltpu.async_copy(table_hbm.at[idx_v.at[j]], rows_v, sem).wait()
        pltpu.sync_copy(rows_v, out_hbm.at[pl.ds(base + j * _CHUNK, _CHUNK)])
        return carry

    lax.fori_loop(0, _NCHUNK, body, 0)


def kernel(x, weight):
    x3 = x.astype(jnp.int32).reshape(_NW, _NCHUNK, _CHUNK)
    out = _embed_sc(x3, weight)
    return out.reshape(BATCH, SEQ, D)


# speedup vs baseline: 5.2890x; 1.0477x over previous
import functools

import jax
import jax.numpy as jnp
from jax import lax
from jax.experimental import pallas as pl
from jax.experimental.pallas import tpu as pltpu
from jax.experimental.pallas import tpu_sc as plsc

PHONEME_SIZE = 1000
D = 64
BATCH = 16384
SEQ = 50

_INFO = plsc.get_sparse_core_info()
_NC = _INFO.num_cores
_NS = _INFO.num_subcores
_NW = _NC * _NS
_CHUNK = 128
_NBUF = 4

_B_TOTAL = BATCH * SEQ
_B_PER_W = _B_TOTAL // _NW
_NCHUNK = _B_PER_W // _CHUNK
_NITER = _NCHUNK // _NBUF


@functools.partial(
    pl.kernel,
    out_type=jax.ShapeDtypeStruct((_B_TOTAL, D), jnp.float32),
    mesh=plsc.VectorSubcoreMesh(core_axis_name="c", subcore_axis_name="s"),
    compiler_params=pltpu.CompilerParams(use_tc_tiling_on_sc=False),
    scratch_types=[
        pltpu.VMEM((_NCHUNK, _CHUNK), jnp.int32),
        pltpu.VMEM((_NBUF, _CHUNK, D), jnp.float32),
    ]
    + [pltpu.SemaphoreType.DMA] * (2 * _NBUF),
)
def _embed_sc(x_hbm, table_hbm, out_hbm, idx_v, rows_v, *sems):
    sem_g = sems[:_NBUF]
    sem_s = sems[_NBUF:]
    wid = lax.axis_index("s") * _NC + lax.axis_index("c")
    base = wid * _B_PER_W
    pltpu.sync_copy(x_hbm.at[wid], idx_v)

    def gather(j, b, sem):
        return pltpu.make_async_copy(table_hbm.at[idx_v.at[j]], rows_v.at[b], sem)

    def scatter(j, b, sem):
        return pltpu.make_async_copy(
            rows_v.at[b], out_hbm.at[pl.ds(base + j * _CHUNK, _CHUNK)], sem
        )

    for b in range(_NBUF):
        gather(b, b, sem_g[b]).start()

    def body(t, carry):
        j0 = t * _NBUF
        for b in range(_NBUF):
            gather(j0 + b, b, sem_g[b]).wait()
            scatter(j0 + b, b, sem_s[b]).start()

        @pl.when(t < _NITER - 1)
        def _():
            for b in range(_NBUF):
                scatter(j0 + b, b, sem_s[b]).wait()
                gather(j0 + _NBUF + b, b, sem_g[b]).start()

        return carry

    lax.fori_loop(0, _NITER, body, 0)

    for b in range(_NBUF):
        scatter(_NCHUNK - _NBUF + b, b, sem_s[b]).wait()


def kernel(x, weight):
    x3 = x.astype(jnp.int32).reshape(_NW, _NCHUNK, _CHUNK)
    out = _embed_sc(x3, weight)
    return out.reshape(BATCH, SEQ, D)
